# baseline (speedup 1.0000x reference)
import functools

import jax
import jax.numpy as jnp
from jax import lax
from jax.experimental import pallas as pl
from jax.experimental.pallas import tpu as pltpu
from jax.experimental.pallas import tpu_sc as plsc

_PARTS = (
    (0, 1, 2, 3, 4),
    (5, 6, 11, 12),
    (5, 7, 9),
    (6, 8, 10),
    (11, 13, 15),
    (12, 14, 16),
    (15, 17, 18, 19),
    (16, 20, 21, 22),
)
_NP = 8
_NB = 16
_H = 384
_W = 384
_NC = 2
_NS = 16
_L = 16
_RH = _H // 2
_R = 8
_NCHUNK = _RH // _R
_PAIRS = _NCHUNK // 2
_WSLICES = _W // _L

_GROUPS = (
    ((0, 2, 3), (0, 1, 2, 3, 4, 5, 6, 7, 8, 9, 10)),
    ((1, 4, 5, 6, 7),
     (5, 6, 11, 12, 13, 14, 15, 16, 17, 18, 19, 20, 21, 22)),
)
_MAXCH = max(len(chs) for _, chs in _GROUPS)


def _sc_call(pred_keypoints):
    mesh = plsc.VectorSubcoreMesh(
        core_axis_name="c", subcore_axis_name="s", num_cores=_NC,
        num_subcores=_NS)

    @functools.partial(
        pl.kernel,
        out_type=jax.ShapeDtypeStruct((_NB, 2, _NP, 3, _L), jnp.float32),
        mesh=mesh,
        scratch_types=[
            pltpu.VMEM((2, _MAXCH, _R, _W), jnp.float32),
            pltpu.VMEM((_NP, 3, _L), jnp.float32),
            pltpu.SemaphoreType.DMA,
            pltpu.SemaphoreType.DMA,
        ],
    )
    def body(kps_hbm, out_hbm, bufs, res, sem_a, sem_b):
        cid = lax.axis_index("c")
        sid = lax.axis_index("s")
        wid = sid * _NC + cid
        b = lax.div(wid, 2)
        half = lax.rem(wid, 2)
        rbase = half * _RH
        lane_f = lax.iota(jnp.int32, _L).astype(jnp.float32)
        sems = (sem_a, sem_b)

        def group_stats(parts_list, channels):
            nch = len(channels)
            cidx = {ch: i for i, ch in enumerate(channels)}
            ng = len(parts_list)

            def issue(ci, slot):
                for i, ch in enumerate(channels):
                    pltpu.async_copy(
                        kps_hbm.at[b, ch, pl.ds(rbase + ci * _R, _R)],
                        bufs.at[slot, i], sems[slot])

            def drain(slot):
                for i in range(nch):
                    pltpu.make_async_copy(
                        kps_hbm.at[b, 0, pl.ds(0, _R)],
                        bufs.at[slot, i], sems[slot]).wait()

            def compute(ci, slot, carry):
                def row_body(r, c2):
                    cnts, sxs, sys_ = list(c2[0]), list(c2[1]), list(c2[2])
                    rowf = (rbase + ci * _R + r).astype(jnp.float32)
                    rowsums = [jnp.zeros((_L,), jnp.float32)] * ng
                    for cc in range(_WSLICES):
                        xs = [bufs[slot, i, r, pl.ds(cc * _L, _L)]
                              for i in range(nch)]
                        colv = lane_f + float(cc * _L)
                        for gi, p in enumerate(parts_list):
                            js = _PARTS[p]
                            s = xs[cidx[js[0]]]
                            for j in js[1:]:
                                s = s + xs[cidx[j]]
                            pos = jnp.where(s > 0.3, 1.0, 0.0)
                            rowsums[gi] = rowsums[gi] + pos
                            sys_[gi] = sys_[gi] + pos * colv
                    for gi in range(ng):
                        cnts[gi] = cnts[gi] + rowsums[gi]
                        sxs[gi] = sxs[gi] + rowf * rowsums[gi]
                    return (tuple(cnts), tuple(sxs), tuple(sys_))

                return lax.fori_loop(0, _R, row_body, carry)

            issue(0, 0)

            def pair_body(cp, carry):
                ci0 = 2 * cp
                issue(ci0 + 1, 1)
                drain(0)
                carry = compute(ci0, 0, carry)

                @pl.when(cp + 1 < _PAIRS)
                def _():
                    issue(ci0 + 2, 0)

                drain(1)
                return compute(ci0 + 1, 1, carry)

            z = jnp.zeros((_L,), jnp.float32)
            zg = (z,) * ng
            cnts, sxs, sys_ = lax.fori_loop(
                0, _PAIRS, pair_body, (zg, zg, zg))
            for gi, p in enumerate(parts_list):
                res[p, 0] = cnts[gi]
                res[p, 1] = sxs[gi]
                res[p, 2] = sys_[gi]

        for parts_list, channels in _GROUPS:
            group_stats(parts_list, channels)
        pltpu.sync_copy(res, out_hbm.at[b, half])

    return body(pred_keypoints)


def _tc_mask_stats(pred_masks):
    def body(m_ref, o_ref):
        x = m_ref[0]
        pos = (x[1:1 + _NP] > 0.5).astype(jnp.float32)
        rows = lax.broadcasted_iota(jnp.int32, (_NP, _H, _W), 1).astype(
            jnp.float32)
        cols = lax.broadcasted_iota(jnp.int32, (_NP, _H, _W), 2).astype(
            jnp.float32)
        cnt = jnp.sum(pos, axis=(1, 2))
        sx = jnp.sum(pos * rows, axis=(1, 2))
        sy = jnp.sum(pos * cols, axis=(1, 2))
        o_ref[0] = jnp.stack([cnt, sx, sy], axis=0)

    return pl.pallas_call(
        body,
        grid=(_NB,),
        in_specs=[pl.BlockSpec((1, 9, _H, _W), lambda b: (b, 0, 0, 0))],
        out_specs=pl.BlockSpec((1, 3, _NP), lambda b: (b, 0, 0)),
        out_shape=jax.ShapeDtypeStruct((_NB, 3, _NP), jnp.float32),
    )(pred_masks)


def _center(cnt, s):
    c = jnp.where(cnt > 0, s / jnp.maximum(cnt, 1.0), 0.0)
    return jnp.where(c > 0, c, 0.0)


@jax.jit
def kernel(pred_masks, pred_keypoints):
    kst = _sc_call(pred_keypoints)
    mst = _tc_mask_stats(pred_masks)
    kst = kst.sum(axis=(1, 4))
    cm, sxm, sym = mst[:, 0].T, mst[:, 1].T, mst[:, 2].T
    ck, sxk, syk = kst[..., 0].T, kst[..., 1].T, kst[..., 2].T
    mcx, mcy = _center(cm, sxm), _center(cm, sym)
    kcx, kcy = _center(ck, sxk), _center(ck, syk)
    code = (mcx == 0) | (mcy == 0) | (kcx == 0) | (kcy == 0)
    valid = (~code).astype(jnp.float32)
    num = jnp.sum(((mcx - kcx) ** 2 + (mcy - kcy) ** 2) * valid)
    den = jnp.maximum(2.0 * jnp.sum(valid), 1.0)
    return 1e-05 * (num / den)
